# baseline (speedup 1.0000x reference)
import functools

import jax
import jax.numpy as jnp
from jax import lax
from jax.experimental import pallas as pl
from jax.experimental.pallas import tpu as pltpu
from jax.experimental.pallas import tpu_sc as plsc

H = 128
N = 20000
N_PAD = 20480
TILE = 2048
GRID = N_PAD // TILE
M = 256
MAX_SELECT = 256
ROWS2D = 8
COLS2D = N_PAD // ROWS2D
IDX_CHUNK = 128


def _sc_gather(table, idx):
    b = idx.shape[0]
    info = plsc.get_sparse_core_info()
    nc, ns = info.num_cores, info.num_subcores
    nw = nc * ns
    b_per_w = b // nw
    chunk = min(IDX_CHUNK, b_per_w)
    n_chunks = b_per_w // chunk
    mesh = plsc.VectorSubcoreMesh(core_axis_name="c", subcore_axis_name="s")

    @functools.partial(
        pl.kernel,
        mesh=mesh,
        out_type=jax.ShapeDtypeStruct((b, H), jnp.float32),
        scratch_types=[
            pltpu.VMEM((b_per_w,), jnp.int32),
            pltpu.VMEM((b_per_w, H), jnp.float32),
            pltpu.SemaphoreType.DMA,
        ],
    )
    def k(table_hbm, idx_hbm, out_hbm, idx_v, rows_v, sem):
        wid = lax.axis_index("s") * nc + lax.axis_index("c")
        base = wid * b_per_w
        pltpu.sync_copy(idx_hbm.at[pl.ds(base, b_per_w)], idx_v)
        handles = []
        for c in range(n_chunks):
            handles.append(
                pltpu.async_copy(
                    table_hbm.at[idx_v.at[pl.ds(c * chunk, chunk)]],
                    rows_v.at[pl.ds(c * chunk, chunk)],
                    sem,
                )
            )
        for h in handles:
            h.wait()
        pltpu.sync_copy(rows_v, out_hbm.at[pl.ds(base, b_per_w)])

    return k(table, idx)


K_PAD = 384


def _mlp_body(m_ref, p_ref, g_ref, xh_ref, xmid_ref, xl_ref, w1_ref,
              b1_ref, w2_ref, b2_ref, w3_ref, b3_ref, out_ref):
    f32 = jnp.float32
    mm = m_ref[0, 0, :]
    onehot = (mm[:, None] ==
              lax.broadcasted_iota(jnp.int32, (TILE, M), 1)).astype(f32)
    xm = (jnp.dot(onehot, xh_ref[...], preferred_element_type=f32)
          + jnp.dot(onehot, xmid_ref[...], preferred_element_type=f32)
          + jnp.dot(onehot, xl_ref[...], preferred_element_type=f32))
    pt = p_ref[0, 0, :]
    x = jnp.concatenate(
        [xm, g_ref[...], pt[:, None], jnp.zeros((TILE, K_PAD - 2 * H - 1), f32)],
        axis=1)
    pre = jnp.dot(x, w1_ref[...], preferred_element_type=f32) + b1_ref[...]
    h1 = jnp.where(pre >= 0, pre, 0.01 * pre)
    pre2 = jnp.dot(h1, w2_ref[...], preferred_element_type=f32) + b2_ref[...]
    h2 = jnp.where(pre2 >= 0, pre2, 0.01 * pre2)
    s = jnp.dot(h2, w3_ref[...], preferred_element_type=f32) + b3_ref[...]
    out_ref[0, 0, :] = s[:, 0]


def _mlp_scores(m3, p3, g, xh, xmid, xl, w1p, b1, w2, b2, w3, b3):
    grid = m3.shape[0]
    full = lambda shape: pl.BlockSpec(shape, lambda i: (0,) * len(shape))
    return pl.pallas_call(
        _mlp_body,
        grid=(grid,),
        in_specs=[
            pl.BlockSpec((1, 1, TILE), lambda i: (i, 0, 0)),
            pl.BlockSpec((1, 1, TILE), lambda i: (i, 0, 0)),
            pl.BlockSpec((TILE, H), lambda i: (i, 0)),
            full((M, H)), full((M, H)), full((M, H)), full((K_PAD, H)),
            full((1, H)), full((H, H)), full((1, H)), full((H, 1)),
            full((1, 1)),
        ],
        out_specs=pl.BlockSpec((1, 1, TILE), lambda i: (i, 0, 0)),
        out_shape=jax.ShapeDtypeStruct((grid, 1, TILE), jnp.float32),
    )(m3, p3, g, xh, xmid, xl, w1p, b1, w2, b2, w3, b3)


def _select_body(s_ref, job_ref, mach_ref, ms_ref, probs_ref, sel_ref):
    ridx = lax.broadcasted_iota(jnp.int32, (ROWS2D, COLS2D), 0)
    cidx = lax.broadcasted_iota(jnp.int32, (ROWS2D, COLS2D), 1)
    gidx = ridx * COLS2D + cidx
    neg = jnp.float32(-jnp.inf)
    s = jnp.where(gidx < N, s_ref[...], neg)
    mx = jnp.max(s)
    e = jnp.exp(s - mx)
    tot = jnp.sum(e)
    probs = e / tot
    probs_ref[...] = probs

    pg_m = (gidx << 8) | mach_ref[...]
    pg_j = (gidx << 13) | job_ref[...]
    ms = ms_ref[0, 0]
    sidx = (lax.broadcasted_iota(jnp.int32, (ROWS2D, 32), 0) * 32 +
            lax.broadcasted_iota(jnp.int32, (ROWS2D, 32), 1))
    big = jnp.int32(2**30)

    def body(i, carry):
        masked, sel = carry
        v = jnp.max(masked)
        found = (v > neg) & (i < ms)
        hit = masked == v
        a = jnp.min(jnp.where(hit, pg_m, big))
        b = jnp.min(jnp.where(hit, pg_j, big))
        idx = a >> 8
        m = a & 255
        j = b & 8191
        sel = jnp.where(found & (sidx == i), idx, sel)
        supp = ((pg_j & 8191) == j) | ((pg_m & 255) == m)
        masked = jnp.where(found & supp, neg, masked)
        return masked, sel

    masked0 = jnp.where(probs > 0.0, probs, neg)
    sel0 = jnp.full((ROWS2D, 32), -1, jnp.int32)
    _, sel = lax.fori_loop(0, MAX_SELECT, body, (masked0, sel0), unroll=8)
    sel_ref[...] = sel


def _select(s2, job2, m2, ms11):
    return pl.pallas_call(
        _select_body,
        out_shape=(
            jax.ShapeDtypeStruct((ROWS2D, COLS2D), jnp.float32),
            jax.ShapeDtypeStruct((ROWS2D, 32), jnp.int32),
        ),
    )(s2, job2, m2, ms11)


def kernel(x_m, x_op, m_ids, op_idxs, proc_times, job_ids, max_select,
           W1, b1, W2, b2, W3, b3):
    pad = N_PAD - N
    i32 = jnp.int32
    opi32 = op_idxs.astype(i32)

    m3 = jnp.concatenate([m_ids.astype(i32), jnp.zeros((pad,), i32)]
                         ).reshape(GRID, 1, TILE)
    p3 = jnp.concatenate([proc_times, jnp.zeros((pad,), jnp.float32)]
                         ).reshape(GRID, 1, TILE)
    w1p = jnp.concatenate([W1, jnp.zeros((K_PAD - 2 * H - 1, H), jnp.float32)])
    xh = x_m.astype(jnp.bfloat16).astype(jnp.float32)
    r1 = x_m - xh
    xmid = r1.astype(jnp.bfloat16).astype(jnp.float32)
    xl = r1 - xmid

    bounds = list(range(0, 20481, 2048))
    parts = []
    for lo, hi_ in zip(bounds[:-1], bounds[1:]):
        glo, ghi = lo // TILE, hi_ // TILE
        if hi_ <= N:
            idx_part = opi32[lo:hi_]
        else:
            idx_part = jnp.concatenate([opi32[lo:], jnp.zeros((pad,), i32)])
        g_p = _sc_gather(x_op, idx_part)
        parts.append(_mlp_scores(m3[glo:ghi], p3[glo:ghi], g_p, xh, xmid, xl,
                                 w1p, b1.reshape(1, H), W2, b2.reshape(1, H),
                                 W3, b3.reshape(1, 1)))

    s2 = jnp.concatenate(parts).reshape(ROWS2D, COLS2D)
    job2 = jnp.concatenate([job_ids.astype(i32), jnp.zeros((pad,), i32)]
                           ).reshape(ROWS2D, COLS2D)
    m2 = m3.reshape(ROWS2D, COLS2D)
    ms11 = jnp.asarray(max_select, i32).reshape(1, 1)
    probs2, sel2 = _select(s2, job2, m2, ms11)
    probs = probs2.reshape(N_PAD)[:N]
    selected = sel2.reshape(MAX_SELECT)
    return probs, selected


# speedup vs baseline: 1.1006x; 1.1006x over previous
import functools

import jax
import jax.numpy as jnp
from jax import lax
from jax.experimental import pallas as pl
from jax.experimental.pallas import tpu as pltpu
from jax.experimental.pallas import tpu_sc as plsc

H = 128
N = 20000
N_PAD = 20480
TILE = 4096
GRID = N_PAD // TILE
M = 256
MAX_SELECT = 256
ROWS2D = 8
COLS2D = N_PAD // ROWS2D
IDX_CHUNK = 128


def _sc_gather(table, idx):
    b = idx.shape[0]
    info = plsc.get_sparse_core_info()
    nc, ns = info.num_cores, info.num_subcores
    nw = nc * ns
    b_per_w = b // nw
    chunk = min(IDX_CHUNK, b_per_w)
    n_chunks = b_per_w // chunk
    mesh = plsc.VectorSubcoreMesh(core_axis_name="c", subcore_axis_name="s")

    @functools.partial(
        pl.kernel,
        mesh=mesh,
        out_type=jax.ShapeDtypeStruct((b, H), jnp.float32),
        scratch_types=[
            pltpu.VMEM((b_per_w,), jnp.int32),
            pltpu.VMEM((b_per_w, H), jnp.float32),
            pltpu.SemaphoreType.DMA,
        ],
    )
    def k(table_hbm, idx_hbm, out_hbm, idx_v, rows_v, sem):
        wid = lax.axis_index("s") * nc + lax.axis_index("c")
        base = wid * b_per_w
        pltpu.sync_copy(idx_hbm.at[pl.ds(base, b_per_w)], idx_v)
        handles = []
        for c in range(n_chunks):
            handles.append(
                pltpu.async_copy(
                    table_hbm.at[idx_v.at[pl.ds(c * chunk, chunk)]],
                    rows_v.at[pl.ds(c * chunk, chunk)],
                    sem,
                )
            )
        for h in handles:
            h.wait()
        pltpu.sync_copy(rows_v, out_hbm.at[pl.ds(base, b_per_w)])

    return k(table, idx)


K_PAD = 384


def _mlp_body(m_ref, p_ref, g_ref, xh_ref, xmid_ref, xl_ref, w1_ref,
              b1_ref, w2_ref, b2_ref, w3_ref, b3_ref, out_ref):
    f32 = jnp.float32
    mm = m_ref[0, 0, :]
    onehot = (mm[:, None] ==
              lax.broadcasted_iota(jnp.int32, (TILE, M), 1)).astype(f32)
    xm = (jnp.dot(onehot, xh_ref[...], preferred_element_type=f32)
          + jnp.dot(onehot, xmid_ref[...], preferred_element_type=f32)
          + jnp.dot(onehot, xl_ref[...], preferred_element_type=f32))
    pt = p_ref[0, 0, :]
    x = jnp.concatenate(
        [xm, g_ref[...], pt[:, None], jnp.zeros((TILE, K_PAD - 2 * H - 1), f32)],
        axis=1)
    pre = jnp.dot(x, w1_ref[...], preferred_element_type=f32) + b1_ref[...]
    h1 = jnp.where(pre >= 0, pre, 0.01 * pre)
    pre2 = jnp.dot(h1, w2_ref[...], preferred_element_type=f32) + b2_ref[...]
    h2 = jnp.where(pre2 >= 0, pre2, 0.01 * pre2)
    s = jnp.dot(h2, w3_ref[...], preferred_element_type=f32) + b3_ref[...]
    out_ref[0, 0, :] = s[:, 0]


def _mlp_scores(m3, p3, g, xh, xmid, xl, w1p, b1, w2, b2, w3, b3):
    grid = m3.shape[0]
    full = lambda shape: pl.BlockSpec(shape, lambda i: (0,) * len(shape))
    return pl.pallas_call(
        _mlp_body,
        grid=(grid,),
        in_specs=[
            pl.BlockSpec((1, 1, TILE), lambda i: (i, 0, 0)),
            pl.BlockSpec((1, 1, TILE), lambda i: (i, 0, 0)),
            pl.BlockSpec((TILE, H), lambda i: (i, 0)),
            full((M, H)), full((M, H)), full((M, H)), full((K_PAD, H)),
            full((1, H)), full((H, H)), full((1, H)), full((H, 1)),
            full((1, 1)),
        ],
        out_specs=pl.BlockSpec((1, 1, TILE), lambda i: (i, 0, 0)),
        out_shape=jax.ShapeDtypeStruct((grid, 1, TILE), jnp.float32),
    )(m3, p3, g, xh, xmid, xl, w1p, b1, w2, b2, w3, b3)


def _select_body(s_ref, job_ref, mach_ref, ms_ref, probs_ref, sel_ref):
    ridx = lax.broadcasted_iota(jnp.int32, (ROWS2D, COLS2D), 0)
    cidx = lax.broadcasted_iota(jnp.int32, (ROWS2D, COLS2D), 1)
    gidx = ridx * COLS2D + cidx
    neg = jnp.float32(-jnp.inf)
    s = jnp.where(gidx < N, s_ref[...], neg)
    mx = jnp.max(s)
    e = jnp.exp(s - mx)
    tot = jnp.sum(e)
    probs = e / tot
    probs_ref[...] = probs

    pg_m = (gidx << 8) | mach_ref[...]
    pg_j = (gidx << 13) | job_ref[...]
    ms = ms_ref[0, 0]
    sidx = (lax.broadcasted_iota(jnp.int32, (ROWS2D, 32), 0) * 32 +
            lax.broadcasted_iota(jnp.int32, (ROWS2D, 32), 1))
    big = jnp.int32(2**30)

    def body(i, carry):
        masked, sel = carry
        v = jnp.max(masked)
        found = (v > neg) & (i < ms)
        hit = masked == v
        a = jnp.min(jnp.where(hit, pg_m, big))
        b = jnp.min(jnp.where(hit, pg_j, big))
        idx = a >> 8
        m = a & 255
        j = b & 8191
        sel = jnp.where(found & (sidx == i), idx, sel)
        supp = ((pg_j & 8191) == j) | ((pg_m & 255) == m)
        masked = jnp.where(found & supp, neg, masked)
        return masked, sel

    masked0 = jnp.where(probs > 0.0, probs, neg)
    sel0 = jnp.full((ROWS2D, 32), -1, jnp.int32)
    _, sel = lax.fori_loop(0, MAX_SELECT, body, (masked0, sel0), unroll=8)
    sel_ref[...] = sel


def _select(s2, job2, m2, ms11):
    return pl.pallas_call(
        _select_body,
        out_shape=(
            jax.ShapeDtypeStruct((ROWS2D, COLS2D), jnp.float32),
            jax.ShapeDtypeStruct((ROWS2D, 32), jnp.int32),
        ),
    )(s2, job2, m2, ms11)


def kernel(x_m, x_op, m_ids, op_idxs, proc_times, job_ids, max_select,
           W1, b1, W2, b2, W3, b3):
    pad = N_PAD - N
    i32 = jnp.int32
    opi32 = op_idxs.astype(i32)

    m3 = jnp.concatenate([m_ids.astype(i32), jnp.zeros((pad,), i32)]
                         ).reshape(GRID, 1, TILE)
    p3 = jnp.concatenate([proc_times, jnp.zeros((pad,), jnp.float32)]
                         ).reshape(GRID, 1, TILE)
    w1p = jnp.concatenate([W1, jnp.zeros((K_PAD - 2 * H - 1, H), jnp.float32)])
    xh = x_m.astype(jnp.bfloat16).astype(jnp.float32)
    r1 = x_m - xh
    xmid = r1.astype(jnp.bfloat16).astype(jnp.float32)
    xl = r1 - xmid

    bounds = [0, 4096, 8192, 12288, 16384, 20480]
    parts = []
    for lo, hi_ in zip(bounds[:-1], bounds[1:]):
        glo, ghi = lo // TILE, hi_ // TILE
        if hi_ <= N:
            idx_part = opi32[lo:hi_]
        else:
            idx_part = jnp.concatenate([opi32[lo:], jnp.zeros((pad,), i32)])
        g_p = _sc_gather(x_op, idx_part)
        parts.append(_mlp_scores(m3[glo:ghi], p3[glo:ghi], g_p, xh, xmid, xl,
                                 w1p, b1.reshape(1, H), W2, b2.reshape(1, H),
                                 W3, b3.reshape(1, 1)))

    s2 = jnp.concatenate(parts).reshape(ROWS2D, COLS2D)
    job2 = jnp.concatenate([job_ids.astype(i32), jnp.zeros((pad,), i32)]
                           ).reshape(ROWS2D, COLS2D)
    m2 = m3.reshape(ROWS2D, COLS2D)
    ms11 = jnp.asarray(max_select, i32).reshape(1, 1)
    probs2, sel2 = _select(s2, job2, m2, ms11)
    probs = probs2.reshape(N_PAD)[:N]
    selected = sel2.reshape(MAX_SELECT)
    return probs, selected
